# Initial kernel scaffold; baseline (speedup 1.0000x reference)
import jax
import jax.numpy as jnp
from jax.experimental import pallas as pl


def kernel(x, w1, b1, wf, bf, w4, b4, w2, b2, wmax, bmax, w3, b3, w3_, b3_):
    raise NotImplementedError("write your pallas kernel here")


# speedup vs baseline: 1.1242x; 1.1242x over previous
import functools

import numpy as np
import jax
import jax.numpy as jnp
from jax.experimental import pallas as pl
from jax.experimental.pallas import tpu as pltpu

_HIGHEST = jax.lax.Precision.HIGHEST
_VMEM_LIMIT = 64 * 1024 * 1024


def _bilinear_matrix_np(out_size, in_size):
    scale = in_size / out_size
    i = np.arange(out_size, dtype=np.float64)
    src = np.maximum((i + 0.5) * scale - 0.5, 0.0)
    i0 = np.clip(np.floor(src).astype(np.int64), 0, in_size - 1)
    i1 = np.minimum(i0 + 1, in_size - 1)
    lam = src - i0
    M = np.zeros((out_size, in_size), np.float32)
    M[np.arange(out_size), i0] += (1.0 - lam)
    M[np.arange(out_size), i1] += lam
    return M


def _esa_kernel(x_ref, wcat_ref, bcat_ref, n2_ref, m2_ref, b2_ref,
                wv_ref, su_ref, r3_ref, lm_ref, bm_ref, l3_ref, b3_ref,
                l3b_ref, b3b_ref, bmt_ref, ah_ref, w4_ref, b4_ref,
                o_ref, *, H, W, f):
    x = x_ref[0]

    y = jnp.dot(wcat_ref[...], x, preferred_element_type=jnp.float32)
    y = y + bcat_ref[...]
    cf = y[f:2 * f, :]

    yt = y[0:f, :].T
    q = jnp.transpose(yt.reshape(H, W, f), (0, 2, 1)).reshape(H * f, W)

    c2 = None
    for kj in range(3):
        t = jnp.dot(q, n2_ref[kj], preferred_element_type=jnp.float32)
        t = jnp.dot(m2_ref[kj], t, preferred_element_type=jnp.float32)
        c2 = t if c2 is None else c2 + t
    c2 = c2 + b2_ref[...]

    wm = None
    for v in range(7):
        t = jnp.dot(c2, wv_ref[v], preferred_element_type=jnp.float32)
        wm = t if wm is None else jnp.maximum(wm, t)
    pm = None
    for u in range(7):
        t = jnp.dot(su_ref[u], wm, preferred_element_type=jnp.float32)
        pm = t if pm is None else jnp.maximum(pm, t)

    def conv3x3(xin, l_ref, b_ref, relu):
        acc = None
        for kj in range(3):
            t = jnp.dot(xin, r3_ref[kj], preferred_element_type=jnp.float32)
            t = jnp.dot(l_ref[kj], t, preferred_element_type=jnp.float32)
            acc = t if acc is None else acc + t
        acc = acc + b_ref[...]
        return jnp.maximum(acc, 0.0) if relu else acc

    v_range = conv3x3(pm, lm_ref, bm_ref, True)
    c3 = conv3x3(v_range, l3_ref, b3_ref, True)
    c3 = conv3x3(c3, l3b_ref, b3b_ref, False)

    tw = jnp.dot(c3, bmt_ref[...], preferred_element_type=jnp.float32)
    r_up = jnp.dot(ah_ref[...], tw, preferred_element_type=jnp.float32)

    s_cm = jnp.transpose(r_up.reshape(H, f, W),
                         (0, 2, 1)).reshape(H * W, f).T

    c4 = jnp.dot(w4_ref[...], cf + s_cm, preferred_element_type=jnp.float32)
    c4 = c4 + b4_ref[...]
    m = pl.reciprocal(1.0 + jnp.exp(-c4), approx=True)
    o_ref[0] = x * m


def kernel(x, w1, b1, wf, bf, w4, b4, w2, b2, wmax, bmax, w3, b3, w3_, b3_):
    N, C, H, W = x.shape
    f = b1.shape[0]
    HW = H * W
    H2 = (H - 3) // 2 + 1
    W2 = (W - 3) // 2 + 1
    Hp = (H2 - 7) // 3 + 1
    Wp = (W2 - 7) // 3 + 1

    wcat_t = jnp.concatenate(
        [w1.T, jnp.dot(w1, wf, precision=_HIGHEST).T], axis=0)
    bcat = jnp.concatenate(
        [b1, jnp.dot(b1, wf, precision=_HIGHEST) + bf]).reshape(2 * f, 1)

    S2 = np.zeros((3, H2, H), np.float32)
    for a in range(3):
        S2[a, np.arange(H2), 2 * np.arange(H2) + a] = 1.0
    m2 = jnp.einsum("aih,ajcg->jighc", jnp.asarray(S2), w2,
                    precision=_HIGHEST).reshape(3, H2 * f, H * f)
    n2 = np.zeros((3, W, W2), np.float32)
    for a in range(3):
        n2[a, 2 * np.arange(W2) + a, np.arange(W2)] = 1.0
    b2_full = jnp.broadcast_to(jnp.tile(b2, H2)[:, None], (H2 * f, W2))

    wv = np.zeros((7, W2, Wp), np.float32)
    for v in range(7):
        wv[v, 3 * np.arange(Wp) + v, np.arange(Wp)] = 1.0
    su = np.zeros((7, Hp * f, H2 * f), np.float32)
    for u in range(7):
        for p in range(Hp):
            for g in range(f):
                su[u, p * f + g, (3 * p + u) * f + g] = 1.0

    r3 = np.zeros((3, Wp, Wp), np.float32)
    for a in range(3):
        for q_ in range(Wp):
            wi = q_ + a - 1
            if 0 <= wi < Wp:
                r3[a, wi, q_] = 1.0
    S3 = np.zeros((3, Hp, Hp), np.float32)
    for a in range(3):
        for p in range(Hp):
            r_ = p + a - 1
            if 0 <= r_ < Hp:
                S3[a, p, r_] = 1.0
    S3j = jnp.asarray(S3)

    def left_taps(w_kkio):
        return jnp.einsum("apr,ajcg->jpgrc", S3j, w_kkio,
                          precision=_HIGHEST).reshape(3, Hp * f, Hp * f)

    lm = left_taps(wmax)
    l3 = left_taps(w3)
    l3b = left_taps(w3_)
    bm_full = jnp.broadcast_to(jnp.tile(bmax, Hp)[:, None], (Hp * f, Wp))
    b3_full = jnp.broadcast_to(jnp.tile(b3, Hp)[:, None], (Hp * f, Wp))
    b3b_full = jnp.broadcast_to(jnp.tile(b3_, Hp)[:, None], (Hp * f, Wp))

    A = _bilinear_matrix_np(H, Hp)
    Bm = _bilinear_matrix_np(W, Wp)
    bmt = Bm.T.copy()
    ah = np.kron(A, np.eye(f, dtype=np.float32))

    x_cm = x.reshape(N, C, HW)
    full = lambda n: (0, 0)
    full3 = lambda n: (0, 0, 0)
    out = pl.pallas_call(
        functools.partial(_esa_kernel, H=H, W=W, f=f),
        out_shape=jax.ShapeDtypeStruct((N, C, HW), jnp.float32),
        grid=(N,),
        in_specs=[
            pl.BlockSpec((1, C, HW), lambda n: (n, 0, 0)),
            pl.BlockSpec((2 * f, C), full),
            pl.BlockSpec((2 * f, 1), full),
            pl.BlockSpec((3, W, W2), full3),
            pl.BlockSpec((3, H2 * f, H * f), full3),
            pl.BlockSpec((H2 * f, W2), full),
            pl.BlockSpec((7, W2, Wp), full3),
            pl.BlockSpec((7, Hp * f, H2 * f), full3),
            pl.BlockSpec((3, Wp, Wp), full3),
            pl.BlockSpec((3, Hp * f, Hp * f), full3),
            pl.BlockSpec((Hp * f, Wp), full),
            pl.BlockSpec((3, Hp * f, Hp * f), full3),
            pl.BlockSpec((Hp * f, Wp), full),
            pl.BlockSpec((3, Hp * f, Hp * f), full3),
            pl.BlockSpec((Hp * f, Wp), full),
            pl.BlockSpec((Wp, W), full),
            pl.BlockSpec((H * f, Hp * f), full),
            pl.BlockSpec((C, f), full),
            pl.BlockSpec((C, 1), full),
        ],
        out_specs=pl.BlockSpec((1, C, HW), lambda n: (n, 0, 0)),
        compiler_params=pltpu.CompilerParams(
            dimension_semantics=("parallel",),
            vmem_limit_bytes=_VMEM_LIMIT),
    )(x_cm, wcat_t, bcat, jnp.asarray(n2), m2, b2_full, jnp.asarray(wv),
      jnp.asarray(su), jnp.asarray(r3), lm, bm_full, l3, b3_full, l3b,
      b3b_full, jnp.asarray(bmt), jnp.asarray(ah), w4.T, b4.reshape(C, 1))
    return out.reshape(N, C, H, W)


# speedup vs baseline: 1.4061x; 1.2507x over previous
import functools

import numpy as np
import jax
import jax.numpy as jnp
from jax.experimental import pallas as pl
from jax.experimental.pallas import tpu as pltpu

_VMEM_LIMIT = 64 * 1024 * 1024


def _bilinear_matrix_np(out_size, in_size):
    scale = in_size / out_size
    i = np.arange(out_size, dtype=np.float64)
    src = np.maximum((i + 0.5) * scale - 0.5, 0.0)
    i0 = np.clip(np.floor(src).astype(np.int64), 0, in_size - 1)
    i1 = np.minimum(i0 + 1, in_size - 1)
    lam = src - i0
    M = np.zeros((out_size, in_size), np.float32)
    M[np.arange(out_size), i0] += (1.0 - lam)
    M[np.arange(out_size), i1] += lam
    return M


def _esa_kernel(x_ref, w1_ref, b1_ref, wf_ref, bf_ref, w4_ref, b4_ref,
                w2_ref, b2_ref, wm_ref, bm_ref, w3_ref, b3_ref,
                w3b_ref, b3b_ref,
                n2_ref, wv_ref, su_ref, r3_ref,
                s2rep_ref, tb1_ref, tb2_ref, s3rep_ref, tc1_ref, tc2_ref,
                uf_ref,
                o_ref,
                wcat_s, w4t_s, b4c_s, m2_s, lm_s, l3_s, l3b_s,
                b2f_s, bmf_s, b3f_s, b3bf_s,
                *, H, W, f, H2, W2, Hp, Wp):
    @pl.when(pl.program_id(0) == 0)
    def _prep():
        w1t = w1_ref[...].T
        wcat_s[0:f, :] = w1t
        wcat_s[f:2 * f, :] = jnp.dot(wf_ref[...].T, w1t,
                                     preferred_element_type=jnp.float32)
        w4t_s[...] = w4_ref[...].T
        bff = jnp.dot(b1_ref[...], wf_ref[...],
                      preferred_element_type=jnp.float32) + bf_ref[...]
        b4c_s[...] = (b4_ref[...] +
                      jnp.dot(bff, w4_ref[...],
                              preferred_element_type=jnp.float32)).T

        for kj in range(3):
            acc = None
            for ki in range(3):
                tile_w = jnp.dot(
                    tb1_ref[...],
                    jnp.dot(w2_ref[ki, kj].T, tb2_ref[...],
                            preferred_element_type=jnp.float32),
                    preferred_element_type=jnp.float32)
                t = s2rep_ref[ki] * tile_w
                acc = t if acc is None else acc + t
            m2_s[kj] = acc

        def left_taps(wref, out_s):
            for kj in range(3):
                acc = None
                for ki in range(3):
                    tile_w = jnp.dot(
                        tc1_ref[...],
                        jnp.dot(wref[ki, kj].T, tc2_ref[...],
                                preferred_element_type=jnp.float32),
                        preferred_element_type=jnp.float32)
                    t = s3rep_ref[ki] * tile_w
                    acc = t if acc is None else acc + t
                out_s[kj] = acc

        left_taps(wm_ref, lm_s)
        left_taps(w3_ref, l3_s)
        left_taps(w3b_ref, l3b_s)

        b2e = b2_ref[...]
        for ki in range(3):
            for kj in range(3):
                b2e = b2e + jnp.dot(b1_ref[...], w2_ref[ki, kj],
                                    preferred_element_type=jnp.float32)
        ones_w2 = jnp.ones((1, W2), jnp.float32)
        b2f_s[...] = jnp.dot(tb1_ref[...],
                             jnp.dot(b2e.T, ones_w2,
                                     preferred_element_type=jnp.float32),
                             preferred_element_type=jnp.float32)
        ones_wp = jnp.ones((1, Wp), jnp.float32)

        def bias_full(bref, out_s):
            out_s[...] = jnp.dot(tc1_ref[...],
                                 jnp.dot(bref[...].T, ones_wp,
                                         preferred_element_type=jnp.float32),
                                 preferred_element_type=jnp.float32)

        bias_full(bm_ref, bmf_s)
        bias_full(b3_ref, b3f_s)
        bias_full(b3b_ref, b3bf_s)

    x = x_ref[0]

    y = jnp.dot(wcat_s[...], x, preferred_element_type=jnp.float32)
    cf = y[f:2 * f, :]

    yt = y[0:f, :].T
    q = jnp.transpose(yt.reshape(H, W, f), (0, 2, 1)).reshape(H * f, W)

    c2 = None
    for kj in range(3):
        t = jnp.dot(q, n2_ref[kj], preferred_element_type=jnp.float32)
        t = jnp.dot(m2_s[kj], t, preferred_element_type=jnp.float32)
        c2 = t if c2 is None else c2 + t
    c2 = c2 + b2f_s[...]

    wm = None
    for v in range(7):
        t = jnp.dot(c2, wv_ref[v], preferred_element_type=jnp.float32)
        wm = t if wm is None else jnp.maximum(wm, t)
    pm = None
    for u in range(7):
        t = jnp.dot(su_ref[u], wm, preferred_element_type=jnp.float32)
        pm = t if pm is None else jnp.maximum(pm, t)

    def conv3x3(xin, l_s, b_s, relu):
        acc = None
        for kj in range(3):
            t = jnp.dot(xin, r3_ref[kj], preferred_element_type=jnp.float32)
            t = jnp.dot(l_s[kj], t, preferred_element_type=jnp.float32)
            acc = t if acc is None else acc + t
        acc = acc + b_s[...]
        return jnp.maximum(acc, 0.0) if relu else acc

    v_range = conv3x3(pm, lm_s, bmf_s, True)
    c3 = conv3x3(v_range, l3_s, b3f_s, True)
    c3 = conv3x3(c3, l3b_s, b3bf_s, False)

    c3cm = jnp.transpose(c3.reshape(Hp, f, Wp),
                         (0, 2, 1)).reshape(Hp * Wp, f).T
    s_cm = jnp.dot(c3cm, uf_ref[...], preferred_element_type=jnp.float32)

    c4 = jnp.dot(w4t_s[...], cf + s_cm, preferred_element_type=jnp.float32)
    c4 = c4 + b4c_s[...]
    m = pl.reciprocal(1.0 + jnp.exp(-c4), approx=True)
    o_ref[0] = x * m


def kernel(x, w1, b1, wf, bf, w4, b4, w2, b2, wmax, bmax, w3, b3, w3_, b3_):
    N, C, H, W = x.shape
    f = b1.shape[0]
    HW = H * W
    H2 = (H - 3) // 2 + 1
    W2 = (W - 3) // 2 + 1
    Hp = (H2 - 7) // 3 + 1
    Wp = (W2 - 7) // 3 + 1

    S2 = np.zeros((3, H2, H), np.float32)
    for a in range(3):
        S2[a, np.arange(H2), 2 * np.arange(H2) + a] = 1.0
    s2rep = np.stack([np.kron(S2[a], np.ones((f, f), np.float32))
                      for a in range(3)])
    tb1 = np.kron(np.ones((H2, 1), np.float32), np.eye(f, dtype=np.float32))
    tb2 = np.kron(np.ones((1, H), np.float32), np.eye(f, dtype=np.float32))
    n2 = np.zeros((3, W, W2), np.float32)
    for a in range(3):
        n2[a, 2 * np.arange(W2) + a, np.arange(W2)] = 1.0

    wv = np.zeros((7, W2, Wp), np.float32)
    for v in range(7):
        wv[v, 3 * np.arange(Wp) + v, np.arange(Wp)] = 1.0
    su = np.zeros((7, Hp * f, H2 * f), np.float32)
    for u in range(7):
        for p in range(Hp):
            for g in range(f):
                su[u, p * f + g, (3 * p + u) * f + g] = 1.0

    r3 = np.zeros((3, Wp, Wp), np.float32)
    for a in range(3):
        for q_ in range(Wp):
            wi = q_ + a - 1
            if 0 <= wi < Wp:
                r3[a, wi, q_] = 1.0
    S3 = np.zeros((3, Hp, Hp), np.float32)
    for a in range(3):
        for p in range(Hp):
            r_ = p + a - 1
            if 0 <= r_ < Hp:
                S3[a, p, r_] = 1.0
    s3rep = np.stack([np.kron(S3[a], np.ones((f, f), np.float32))
                      for a in range(3)])
    tc1 = np.kron(np.ones((Hp, 1), np.float32), np.eye(f, dtype=np.float32))
    tc2 = np.kron(np.ones((1, Hp), np.float32), np.eye(f, dtype=np.float32))

    A = _bilinear_matrix_np(H, Hp)
    Bm = _bilinear_matrix_np(W, Wp)
    ufull = np.kron(A, Bm).T.copy()

    x_cm = x.reshape(N, C, HW)
    full = lambda n: (0, 0)
    full3 = lambda n: (0, 0, 0)
    full4 = lambda n: (0, 0, 0, 0)
    row = lambda v: v.reshape(1, -1)
    out = pl.pallas_call(
        functools.partial(_esa_kernel, H=H, W=W, f=f,
                          H2=H2, W2=W2, Hp=Hp, Wp=Wp),
        out_shape=jax.ShapeDtypeStruct((N, C, HW), jnp.float32),
        grid=(N,),
        in_specs=[
            pl.BlockSpec((1, C, HW), lambda n: (n, 0, 0)),
            pl.BlockSpec((C, f), full),
            pl.BlockSpec((1, f), full),
            pl.BlockSpec((f, f), full),
            pl.BlockSpec((1, f), full),
            pl.BlockSpec((f, C), full),
            pl.BlockSpec((1, C), full),
            pl.BlockSpec((3, 3, f, f), full4),
            pl.BlockSpec((1, f), full),
            pl.BlockSpec((3, 3, f, f), full4),
            pl.BlockSpec((1, f), full),
            pl.BlockSpec((3, 3, f, f), full4),
            pl.BlockSpec((1, f), full),
            pl.BlockSpec((3, 3, f, f), full4),
            pl.BlockSpec((1, f), full),
            pl.BlockSpec((3, W, W2), full3),
            pl.BlockSpec((7, W2, Wp), full3),
            pl.BlockSpec((7, Hp * f, H2 * f), full3),
            pl.BlockSpec((3, Wp, Wp), full3),
            pl.BlockSpec((3, H2 * f, H * f), full3),
            pl.BlockSpec((H2 * f, f), full),
            pl.BlockSpec((f, H * f), full),
            pl.BlockSpec((3, Hp * f, Hp * f), full3),
            pl.BlockSpec((Hp * f, f), full),
            pl.BlockSpec((f, Hp * f), full),
            pl.BlockSpec((Hp * Wp, HW), full),
        ],
        out_specs=pl.BlockSpec((1, C, HW), lambda n: (n, 0, 0)),
        scratch_shapes=[
            pltpu.VMEM((2 * f, C), jnp.float32),
            pltpu.VMEM((C, f), jnp.float32),
            pltpu.VMEM((C, 1), jnp.float32),
            pltpu.VMEM((3, H2 * f, H * f), jnp.float32),
            pltpu.VMEM((3, Hp * f, Hp * f), jnp.float32),
            pltpu.VMEM((3, Hp * f, Hp * f), jnp.float32),
            pltpu.VMEM((3, Hp * f, Hp * f), jnp.float32),
            pltpu.VMEM((H2 * f, W2), jnp.float32),
            pltpu.VMEM((Hp * f, Wp), jnp.float32),
            pltpu.VMEM((Hp * f, Wp), jnp.float32),
            pltpu.VMEM((Hp * f, Wp), jnp.float32),
        ],
        compiler_params=pltpu.CompilerParams(
            dimension_semantics=("arbitrary",),
            vmem_limit_bytes=_VMEM_LIMIT),
    )(x_cm, w1, row(b1), wf, row(bf), w4, row(b4), w2, row(b2),
      wmax, row(bmax), w3, row(b3), w3_, row(b3_),
      jnp.asarray(n2), jnp.asarray(wv), jnp.asarray(su), jnp.asarray(r3),
      jnp.asarray(s2rep), jnp.asarray(tb1), jnp.asarray(tb2),
      jnp.asarray(s3rep), jnp.asarray(tc1), jnp.asarray(tc2),
      jnp.asarray(ufull))
    return out.reshape(N, C, H, W)


# speedup vs baseline: 2.1347x; 1.5182x over previous
import functools

import numpy as np
import jax
import jax.numpy as jnp
from jax.experimental import pallas as pl
from jax.experimental.pallas import tpu as pltpu

_VMEM_LIMIT = 64 * 1024 * 1024


def _bilinear_matrix_np(out_size, in_size):
    scale = in_size / out_size
    i = np.arange(out_size, dtype=np.float64)
    src = np.maximum((i + 0.5) * scale - 0.5, 0.0)
    i0 = np.clip(np.floor(src).astype(np.int64), 0, in_size - 1)
    i1 = np.minimum(i0 + 1, in_size - 1)
    lam = src - i0
    M = np.zeros((out_size, in_size), np.float32)
    M[np.arange(out_size), i0] += (1.0 - lam)
    M[np.arange(out_size), i1] += lam
    return M


def _esa_kernel(x_ref, w1_ref, b1_ref, wf_ref, bf_ref, w4_ref, b4_ref,
                w2_ref, b2_ref, wm_ref, bm_ref, w3_ref, b3_ref,
                w3b_ref, b3b_ref,
                n2_ref, wv_ref, su_ref, r3_ref,
                s2rep_ref, tb1_ref, tb2_ref, s3rep_ref, tc1_ref, tc2_ref,
                uf_ref,
                o_ref,
                wcat_s, w4t_s, b4c_s, m2_s, lm_s, l3_s, l3b_s,
                b2f_s, bmf_s, b3f_s, b3bf_s,
                *, H, W, f, H2, W2, Hp, Wp):
    @pl.when(pl.program_id(0) == 0)
    def _prep():
        w1t = w1_ref[...].T
        wcat_s[0:f, :] = w1t
        wcat_s[f:2 * f, :] = jnp.dot(wf_ref[...].T, w1t,
                                     preferred_element_type=jnp.float32)
        w4t_s[...] = w4_ref[...].T
        bff = jnp.dot(b1_ref[...], wf_ref[...],
                      preferred_element_type=jnp.float32) + bf_ref[...]
        b4c_s[...] = (b4_ref[...] +
                      jnp.dot(bff, w4_ref[...],
                              preferred_element_type=jnp.float32)).T

        for kj in range(3):
            acc = None
            for ki in range(3):
                tile_w = jnp.dot(
                    tb1_ref[...],
                    jnp.dot(w2_ref[ki, kj].T, tb2_ref[...],
                            preferred_element_type=jnp.float32),
                    preferred_element_type=jnp.float32)
                t = s2rep_ref[ki] * tile_w
                acc = t if acc is None else acc + t
            m2_s[kj] = acc

        def left_taps(wref, out_s):
            for kj in range(3):
                acc = None
                for ki in range(3):
                    tile_w = jnp.dot(
                        tc1_ref[...],
                        jnp.dot(wref[ki, kj].T, tc2_ref[...],
                                preferred_element_type=jnp.float32),
                        preferred_element_type=jnp.float32)
                    t = s3rep_ref[ki] * tile_w
                    acc = t if acc is None else acc + t
                out_s[kj] = acc

        left_taps(wm_ref, lm_s)
        left_taps(w3_ref, l3_s)
        left_taps(w3b_ref, l3b_s)

        b2e = b2_ref[...]
        for ki in range(3):
            for kj in range(3):
                b2e = b2e + jnp.dot(b1_ref[...], w2_ref[ki, kj],
                                    preferred_element_type=jnp.float32)
        ones_w2 = jnp.ones((1, W2), jnp.float32)
        b2f_s[...] = jnp.dot(tb1_ref[...],
                             jnp.dot(b2e.T, ones_w2,
                                     preferred_element_type=jnp.float32),
                             preferred_element_type=jnp.float32)
        ones_wp = jnp.ones((1, Wp), jnp.float32)

        def bias_full(bref, out_s):
            out_s[...] = jnp.dot(tc1_ref[...],
                                 jnp.dot(bref[...].T, ones_wp,
                                         preferred_element_type=jnp.float32),
                                 preferred_element_type=jnp.float32)

        bias_full(bm_ref, bmf_s)
        bias_full(b3_ref, b3f_s)
        bias_full(b3b_ref, b3bf_s)

    x = x_ref[0].reshape(x_ref.shape[1], x_ref.shape[2] * x_ref.shape[3])

    y = jnp.dot(wcat_s[...], x, preferred_element_type=jnp.float32)
    cf = y[f:2 * f, :]

    yt = y[0:f, :].T
    q = jnp.transpose(yt.reshape(H, W, f), (0, 2, 1)).reshape(H * f, W)

    c2 = None
    for kj in range(3):
        t = jnp.dot(q, n2_ref[kj], preferred_element_type=jnp.float32)
        t = jnp.dot(m2_s[kj], t, preferred_element_type=jnp.float32)
        c2 = t if c2 is None else c2 + t
    c2 = c2 + b2f_s[...]

    wm = None
    for v in range(7):
        t = jnp.dot(c2, wv_ref[v], preferred_element_type=jnp.float32)
        wm = t if wm is None else jnp.maximum(wm, t)
    pm = None
    for u in range(7):
        t = jnp.dot(su_ref[u], wm, preferred_element_type=jnp.float32)
        pm = t if pm is None else jnp.maximum(pm, t)

    def conv3x3(xin, l_s, b_s, relu):
        acc = None
        for kj in range(3):
            t = jnp.dot(xin, r3_ref[kj], preferred_element_type=jnp.float32)
            t = jnp.dot(l_s[kj], t, preferred_element_type=jnp.float32)
            acc = t if acc is None else acc + t
        acc = acc + b_s[...]
        return jnp.maximum(acc, 0.0) if relu else acc

    v_range = conv3x3(pm, lm_s, bmf_s, True)
    c3 = conv3x3(v_range, l3_s, b3f_s, True)
    c3 = conv3x3(c3, l3b_s, b3bf_s, False)

    c3cm = jnp.transpose(c3.reshape(Hp, f, Wp),
                         (0, 2, 1)).reshape(Hp * Wp, f).T
    s_cm = jnp.dot(c3cm, uf_ref[...], preferred_element_type=jnp.float32)

    c4 = jnp.dot(w4t_s[...], cf + s_cm, preferred_element_type=jnp.float32)
    c4 = c4 + b4c_s[...]
    m = pl.reciprocal(1.0 + jnp.exp(-c4), approx=True)
    o_ref[0] = (x * m).reshape(o_ref.shape[1:])


def kernel(x, w1, b1, wf, bf, w4, b4, w2, b2, wmax, bmax, w3, b3, w3_, b3_):
    N, C, H, W = x.shape
    f = b1.shape[0]
    HW = H * W
    H2 = (H - 3) // 2 + 1
    W2 = (W - 3) // 2 + 1
    Hp = (H2 - 7) // 3 + 1
    Wp = (W2 - 7) // 3 + 1

    S2 = np.zeros((3, H2, H), np.float32)
    for a in range(3):
        S2[a, np.arange(H2), 2 * np.arange(H2) + a] = 1.0
    s2rep = np.stack([np.kron(S2[a], np.ones((f, f), np.float32))
                      for a in range(3)])
    tb1 = np.kron(np.ones((H2, 1), np.float32), np.eye(f, dtype=np.float32))
    tb2 = np.kron(np.ones((1, H), np.float32), np.eye(f, dtype=np.float32))
    n2 = np.zeros((3, W, W2), np.float32)
    for a in range(3):
        n2[a, 2 * np.arange(W2) + a, np.arange(W2)] = 1.0

    wv = np.zeros((7, W2, Wp), np.float32)
    for v in range(7):
        wv[v, 3 * np.arange(Wp) + v, np.arange(Wp)] = 1.0
    su = np.zeros((7, Hp * f, H2 * f), np.float32)
    for u in range(7):
        for p in range(Hp):
            for g in range(f):
                su[u, p * f + g, (3 * p + u) * f + g] = 1.0

    r3 = np.zeros((3, Wp, Wp), np.float32)
    for a in range(3):
        for q_ in range(Wp):
            wi = q_ + a - 1
            if 0 <= wi < Wp:
                r3[a, wi, q_] = 1.0
    S3 = np.zeros((3, Hp, Hp), np.float32)
    for a in range(3):
        for p in range(Hp):
            r_ = p + a - 1
            if 0 <= r_ < Hp:
                S3[a, p, r_] = 1.0
    s3rep = np.stack([np.kron(S3[a], np.ones((f, f), np.float32))
                      for a in range(3)])
    tc1 = np.kron(np.ones((Hp, 1), np.float32), np.eye(f, dtype=np.float32))
    tc2 = np.kron(np.ones((1, Hp), np.float32), np.eye(f, dtype=np.float32))

    A = _bilinear_matrix_np(H, Hp)
    Bm = _bilinear_matrix_np(W, Wp)
    ufull = np.kron(A, Bm).T.copy()

    full = lambda n: (0, 0)
    full3 = lambda n: (0, 0, 0)
    full4 = lambda n: (0, 0, 0, 0)
    row = lambda v: v.reshape(1, -1)
    out = pl.pallas_call(
        functools.partial(_esa_kernel, H=H, W=W, f=f,
                          H2=H2, W2=W2, Hp=Hp, Wp=Wp),
        out_shape=jax.ShapeDtypeStruct((N, C, H, W), jnp.float32),
        grid=(N,),
        in_specs=[
            pl.BlockSpec((1, C, H, W), lambda n: (n, 0, 0, 0)),
            pl.BlockSpec((C, f), full),
            pl.BlockSpec((1, f), full),
            pl.BlockSpec((f, f), full),
            pl.BlockSpec((1, f), full),
            pl.BlockSpec((f, C), full),
            pl.BlockSpec((1, C), full),
            pl.BlockSpec((3, 3, f, f), full4),
            pl.BlockSpec((1, f), full),
            pl.BlockSpec((3, 3, f, f), full4),
            pl.BlockSpec((1, f), full),
            pl.BlockSpec((3, 3, f, f), full4),
            pl.BlockSpec((1, f), full),
            pl.BlockSpec((3, 3, f, f), full4),
            pl.BlockSpec((1, f), full),
            pl.BlockSpec((3, W, W2), full3),
            pl.BlockSpec((7, W2, Wp), full3),
            pl.BlockSpec((7, Hp * f, H2 * f), full3),
            pl.BlockSpec((3, Wp, Wp), full3),
            pl.BlockSpec((3, H2 * f, H * f), full3),
            pl.BlockSpec((H2 * f, f), full),
            pl.BlockSpec((f, H * f), full),
            pl.BlockSpec((3, Hp * f, Hp * f), full3),
            pl.BlockSpec((Hp * f, f), full),
            pl.BlockSpec((f, Hp * f), full),
            pl.BlockSpec((Hp * Wp, HW), full),
        ],
        out_specs=pl.BlockSpec((1, C, H, W), lambda n: (n, 0, 0, 0)),
        scratch_shapes=[
            pltpu.VMEM((2 * f, C), jnp.float32),
            pltpu.VMEM((C, f), jnp.float32),
            pltpu.VMEM((C, 1), jnp.float32),
            pltpu.VMEM((3, H2 * f, H * f), jnp.float32),
            pltpu.VMEM((3, Hp * f, Hp * f), jnp.float32),
            pltpu.VMEM((3, Hp * f, Hp * f), jnp.float32),
            pltpu.VMEM((3, Hp * f, Hp * f), jnp.float32),
            pltpu.VMEM((H2 * f, W2), jnp.float32),
            pltpu.VMEM((Hp * f, Wp), jnp.float32),
            pltpu.VMEM((Hp * f, Wp), jnp.float32),
            pltpu.VMEM((Hp * f, Wp), jnp.float32),
        ],
        compiler_params=pltpu.CompilerParams(
            dimension_semantics=("arbitrary",),
            vmem_limit_bytes=_VMEM_LIMIT),
    )(x, w1, row(b1), wf, row(bf), w4, row(b4), w2, row(b2),
      wmax, row(bmax), w3, row(b3), w3_, row(b3_),
      jnp.asarray(n2), jnp.asarray(wv), jnp.asarray(su), jnp.asarray(r3),
      jnp.asarray(s2rep), jnp.asarray(tb1), jnp.asarray(tb2),
      jnp.asarray(s3rep), jnp.asarray(tc1), jnp.asarray(tc2),
      jnp.asarray(ufull))
    return out


# speedup vs baseline: 2.3861x; 1.1177x over previous
import functools

import numpy as np
import jax
import jax.numpy as jnp
from jax.experimental import pallas as pl
from jax.experimental.pallas import tpu as pltpu

_VMEM_LIMIT = 64 * 1024 * 1024


def _bilinear_matrix_np(out_size, in_size):
    scale = in_size / out_size
    i = np.arange(out_size, dtype=np.float64)
    src = np.maximum((i + 0.5) * scale - 0.5, 0.0)
    i0 = np.clip(np.floor(src).astype(np.int64), 0, in_size - 1)
    i1 = np.minimum(i0 + 1, in_size - 1)
    lam = src - i0
    M = np.zeros((out_size, in_size), np.float32)
    M[np.arange(out_size), i0] += (1.0 - lam)
    M[np.arange(out_size), i1] += lam
    return M


def _esa_kernel(x_ref, w1_ref, b1_ref, wf_ref, bf_ref, w4_ref, b4_ref,
                w2_ref, b2_ref, wm_ref, bm_ref, w3_ref, b3_ref,
                w3b_ref, b3b_ref,
                n2_ref, wv_ref, su_ref, r3_ref,
                s2rep_ref, tb1_ref, tb2_ref, s3rep_ref, tc1_ref, tc2_ref,
                uf_ref,
                o_ref,
                wcat_s, w4t_s, b4c_s, m2_s, lm_s, l3_s, l3b_s,
                b2f_s, bmf_s, b3f_s, b3bf_s,
                *, H, W, f, H2, W2, Hp, Wp):
    @pl.when(pl.program_id(0) == 0)
    def _prep():
        w1t = w1_ref[...].T
        wcat_s[0:f, :] = w1t
        wcat_s[f:2 * f, :] = jnp.dot(wf_ref[...].T, w1t,
                                     preferred_element_type=jnp.float32)
        w4t_s[...] = w4_ref[...].T
        bff = jnp.dot(b1_ref[...], wf_ref[...],
                      preferred_element_type=jnp.float32) + bf_ref[...]
        b4c_s[...] = (b4_ref[...] +
                      jnp.dot(bff, w4_ref[...],
                              preferred_element_type=jnp.float32)).T

        for kj in range(3):
            acc = None
            for ki in range(3):
                tile_w = jnp.dot(
                    tb1_ref[...],
                    jnp.dot(w2_ref[ki, kj].T, tb2_ref[...],
                            preferred_element_type=jnp.float32),
                    preferred_element_type=jnp.float32)
                t = s2rep_ref[ki] * tile_w
                acc = t if acc is None else acc + t
            m2_s[kj] = acc

        def left_taps(wref, out_s):
            for kj in range(3):
                acc = None
                for ki in range(3):
                    tile_w = jnp.dot(
                        tc1_ref[...],
                        jnp.dot(wref[ki, kj].T, tc2_ref[...],
                                preferred_element_type=jnp.float32),
                        preferred_element_type=jnp.float32)
                    t = s3rep_ref[ki] * tile_w
                    acc = t if acc is None else acc + t
                out_s[kj] = acc

        left_taps(wm_ref, lm_s)
        left_taps(w3_ref, l3_s)
        left_taps(w3b_ref, l3b_s)

        b2e = b2_ref[...]
        for ki in range(3):
            for kj in range(3):
                b2e = b2e + jnp.dot(b1_ref[...], w2_ref[ki, kj],
                                    preferred_element_type=jnp.float32)
        ones_w2 = jnp.ones((1, W2), jnp.float32)
        b2f_s[...] = jnp.dot(tb1_ref[...],
                             jnp.dot(b2e.T, ones_w2,
                                     preferred_element_type=jnp.float32),
                             preferred_element_type=jnp.float32)
        ones_wp = jnp.ones((1, Wp), jnp.float32)

        def bias_full(bref, out_s):
            out_s[...] = jnp.dot(tc1_ref[...],
                                 jnp.dot(bref[...].T, ones_wp,
                                         preferred_element_type=jnp.float32),
                                 preferred_element_type=jnp.float32)

        bias_full(bm_ref, bmf_s)
        bias_full(b3_ref, b3f_s)
        bias_full(b3b_ref, b3bf_s)

    for b in range(x_ref.shape[0]):
        _one_image(b, x_ref, o_ref, wcat_s, w4t_s, b4c_s, m2_s, lm_s, l3_s,
                   l3b_s, b2f_s, bmf_s, b3f_s, b3bf_s,
                   n2_ref, wv_ref, su_ref, r3_ref, uf_ref, H, W, f, Hp, Wp)


def _one_image(b, x_ref, o_ref, wcat_s, w4t_s, b4c_s, m2_s, lm_s, l3_s,
               l3b_s, b2f_s, bmf_s, b3f_s, b3bf_s,
               n2_ref, wv_ref, su_ref, r3_ref, uf_ref, H, W, f, Hp, Wp):
    x = x_ref[b].reshape(x_ref.shape[1], x_ref.shape[2] * x_ref.shape[3])

    y = jnp.dot(wcat_s[...], x, preferred_element_type=jnp.float32)
    cf = y[f:2 * f, :]

    yt = y[0:f, :].T
    q = jnp.transpose(yt.reshape(H, W, f), (0, 2, 1)).reshape(H * f, W)

    c2 = None
    for kj in range(3):
        t = jnp.dot(q, n2_ref[kj], preferred_element_type=jnp.float32)
        t = jnp.dot(m2_s[kj], t, preferred_element_type=jnp.float32)
        c2 = t if c2 is None else c2 + t
    c2 = c2 + b2f_s[...]

    wm = None
    for v in range(7):
        t = jnp.dot(c2, wv_ref[v], preferred_element_type=jnp.float32)
        wm = t if wm is None else jnp.maximum(wm, t)
    pm = None
    for u in range(7):
        t = jnp.dot(su_ref[u], wm, preferred_element_type=jnp.float32)
        pm = t if pm is None else jnp.maximum(pm, t)

    def conv3x3(xin, l_s, b_s, relu):
        acc = None
        for kj in range(3):
            t = jnp.dot(xin, r3_ref[kj], preferred_element_type=jnp.float32)
            t = jnp.dot(l_s[kj], t, preferred_element_type=jnp.float32)
            acc = t if acc is None else acc + t
        acc = acc + b_s[...]
        return jnp.maximum(acc, 0.0) if relu else acc

    v_range = conv3x3(pm, lm_s, bmf_s, True)
    c3 = conv3x3(v_range, l3_s, b3f_s, True)
    c3 = conv3x3(c3, l3b_s, b3bf_s, False)

    c3cm = jnp.transpose(c3.reshape(Hp, f, Wp),
                         (0, 2, 1)).reshape(Hp * Wp, f).T
    s_cm = jnp.dot(c3cm, uf_ref[...], preferred_element_type=jnp.float32)

    c4 = jnp.dot(w4t_s[...], cf + s_cm, preferred_element_type=jnp.float32)
    c4 = c4 + b4c_s[...]
    m = pl.reciprocal(1.0 + jnp.exp(-c4), approx=True)
    o_ref[b] = (x * m).reshape(o_ref.shape[1:])


def kernel(x, w1, b1, wf, bf, w4, b4, w2, b2, wmax, bmax, w3, b3, w3_, b3_):
    N, C, H, W = x.shape
    f = b1.shape[0]
    HW = H * W
    H2 = (H - 3) // 2 + 1
    W2 = (W - 3) // 2 + 1
    Hp = (H2 - 7) // 3 + 1
    Wp = (W2 - 7) // 3 + 1

    S2 = np.zeros((3, H2, H), np.float32)
    for a in range(3):
        S2[a, np.arange(H2), 2 * np.arange(H2) + a] = 1.0
    s2rep = np.stack([np.kron(S2[a], np.ones((f, f), np.float32))
                      for a in range(3)])
    tb1 = np.kron(np.ones((H2, 1), np.float32), np.eye(f, dtype=np.float32))
    tb2 = np.kron(np.ones((1, H), np.float32), np.eye(f, dtype=np.float32))
    n2 = np.zeros((3, W, W2), np.float32)
    for a in range(3):
        n2[a, 2 * np.arange(W2) + a, np.arange(W2)] = 1.0

    wv = np.zeros((7, W2, Wp), np.float32)
    for v in range(7):
        wv[v, 3 * np.arange(Wp) + v, np.arange(Wp)] = 1.0
    su = np.zeros((7, Hp * f, H2 * f), np.float32)
    for u in range(7):
        for p in range(Hp):
            for g in range(f):
                su[u, p * f + g, (3 * p + u) * f + g] = 1.0

    r3 = np.zeros((3, Wp, Wp), np.float32)
    for a in range(3):
        for q_ in range(Wp):
            wi = q_ + a - 1
            if 0 <= wi < Wp:
                r3[a, wi, q_] = 1.0
    S3 = np.zeros((3, Hp, Hp), np.float32)
    for a in range(3):
        for p in range(Hp):
            r_ = p + a - 1
            if 0 <= r_ < Hp:
                S3[a, p, r_] = 1.0
    s3rep = np.stack([np.kron(S3[a], np.ones((f, f), np.float32))
                      for a in range(3)])
    tc1 = np.kron(np.ones((Hp, 1), np.float32), np.eye(f, dtype=np.float32))
    tc2 = np.kron(np.ones((1, Hp), np.float32), np.eye(f, dtype=np.float32))

    A = _bilinear_matrix_np(H, Hp)
    Bm = _bilinear_matrix_np(W, Wp)
    ufull = np.kron(A, Bm).T.copy()

    full = lambda n: (0, 0)
    full3 = lambda n: (0, 0, 0)
    full4 = lambda n: (0, 0, 0, 0)
    row = lambda v: v.reshape(1, -1)
    for B in (4, 3, 2, 1):
        if N % B == 0:
            break
    out = pl.pallas_call(
        functools.partial(_esa_kernel, H=H, W=W, f=f,
                          H2=H2, W2=W2, Hp=Hp, Wp=Wp),
        out_shape=jax.ShapeDtypeStruct((N, C, H, W), jnp.float32),
        grid=(N // B,),
        in_specs=[
            pl.BlockSpec((B, C, H, W), lambda n: (n, 0, 0, 0)),
            pl.BlockSpec((C, f), full),
            pl.BlockSpec((1, f), full),
            pl.BlockSpec((f, f), full),
            pl.BlockSpec((1, f), full),
            pl.BlockSpec((f, C), full),
            pl.BlockSpec((1, C), full),
            pl.BlockSpec((3, 3, f, f), full4),
            pl.BlockSpec((1, f), full),
            pl.BlockSpec((3, 3, f, f), full4),
            pl.BlockSpec((1, f), full),
            pl.BlockSpec((3, 3, f, f), full4),
            pl.BlockSpec((1, f), full),
            pl.BlockSpec((3, 3, f, f), full4),
            pl.BlockSpec((1, f), full),
            pl.BlockSpec((3, W, W2), full3),
            pl.BlockSpec((7, W2, Wp), full3),
            pl.BlockSpec((7, Hp * f, H2 * f), full3),
            pl.BlockSpec((3, Wp, Wp), full3),
            pl.BlockSpec((3, H2 * f, H * f), full3),
            pl.BlockSpec((H2 * f, f), full),
            pl.BlockSpec((f, H * f), full),
            pl.BlockSpec((3, Hp * f, Hp * f), full3),
            pl.BlockSpec((Hp * f, f), full),
            pl.BlockSpec((f, Hp * f), full),
            pl.BlockSpec((Hp * Wp, HW), full),
        ],
        out_specs=pl.BlockSpec((B, C, H, W), lambda n: (n, 0, 0, 0)),
        scratch_shapes=[
            pltpu.VMEM((2 * f, C), jnp.float32),
            pltpu.VMEM((C, f), jnp.float32),
            pltpu.VMEM((C, 1), jnp.float32),
            pltpu.VMEM((3, H2 * f, H * f), jnp.float32),
            pltpu.VMEM((3, Hp * f, Hp * f), jnp.float32),
            pltpu.VMEM((3, Hp * f, Hp * f), jnp.float32),
            pltpu.VMEM((3, Hp * f, Hp * f), jnp.float32),
            pltpu.VMEM((H2 * f, W2), jnp.float32),
            pltpu.VMEM((Hp * f, Wp), jnp.float32),
            pltpu.VMEM((Hp * f, Wp), jnp.float32),
            pltpu.VMEM((Hp * f, Wp), jnp.float32),
        ],
        compiler_params=pltpu.CompilerParams(
            dimension_semantics=("arbitrary",),
            vmem_limit_bytes=_VMEM_LIMIT),
    )(x, w1, row(b1), wf, row(bf), w4, row(b4), w2, row(b2),
      wmax, row(bmax), w3, row(b3), w3_, row(b3_),
      jnp.asarray(n2), jnp.asarray(wv), jnp.asarray(su), jnp.asarray(r3),
      jnp.asarray(s2rep), jnp.asarray(tb1), jnp.asarray(tb2),
      jnp.asarray(s3rep), jnp.asarray(tc1), jnp.asarray(tc2),
      jnp.asarray(ufull))
    return out
